# baseline (speedup 1.0000x reference)
import functools

import jax
import jax.numpy as jnp
from jax import lax
from jax.experimental import pallas as pl
from jax.experimental.pallas import tpu as pltpu
from jax.experimental.pallas import tpu_sc as plsc

K = 8
HID = 512
DIM = 128
SEQ = 50
REC = 20
N_ITEMS = 100000
KM_ITERS = 5

_B = 2000
_NB = N_ITEMS // _B
_NPAD = 102400
_PROWS = _NPAD // 128

_DEF = lax.Precision.DEFAULT
_HI = lax.Precision.HIGHEST


def _dot(a, b, prec=_DEF):
    return lax.dot_general(a, b, (((1,), (0,)), ((), ())), precision=prec,
                           preferred_element_type=jnp.float32)


def _dot_rt(a, b, prec=_DEF):
    return lax.dot_general(a, b, (((1,), (1,)), ((), ())), precision=prec,
                           preferred_element_type=jnp.float32)


def _kmeans_buckets(items):
    cent = items[:K]
    for _ in range(KM_ITERS):
        d2 = (jnp.sum(items * items, 1, keepdims=True)
              - 2.0 * items @ cent.T
              + jnp.sum(cent * cent, 1)[None, :])
        lab = jnp.argmin(d2, axis=1)
        sums = jax.ops.segment_sum(items, lab, num_segments=K)
        oh = (lab[:, None] == jnp.arange(K)[None, :]).astype(jnp.float32)
        cnt = jnp.sum(oh, axis=0)
        cent = sums / jnp.maximum(cnt, 1.0)[:, None]
    return cent


def _prompt_body(user_ref, cent_ref, pe_ref, out_ref):
    u = user_ref[...]
    c = cent_ref[...]
    s = _dot_rt(u, c)
    bias = lax.dot_general(jnp.ones((1, DIM), jnp.float32), c * c,
                           (((1,), (1,)), ((), ())), precision=_HI,
                           preferred_element_type=jnp.float32)
    xx = jnp.sum(u * u, axis=1, keepdims=True)
    d = (xx - 2.0 * s) + bias
    dmin = jnp.min(d, axis=1, keepdims=True)
    ki = lax.broadcasted_iota(jnp.int32, (SEQ, K), 1)
    idx = jnp.min(jnp.where(d <= dmin, ki, K), axis=1, keepdims=True)
    pr = jnp.zeros((SEQ, DIM), jnp.float32)
    for k in range(K):
        pr = jnp.where(idx == k, pe_ref[k:k + 1, :], pr)
    out_ref[:, :DIM] = u
    out_ref[:, DIM:] = pr


def _prompted_user(user, cent, prompt_emb, interpret=False):
    return pl.pallas_call(
        _prompt_body,
        out_shape=jax.ShapeDtypeStruct((SEQ, 2 * DIM), jnp.float32),
        interpret=interpret,
    )(user, cent, prompt_emb)


def _user_mlp_body(pu_ref, w1_ref, b1_ref, w2_ref, b2_ref, out_ref):
    h = jnp.maximum(_dot(pu_ref[...], w1_ref[...]) + b1_ref[...], 0.0)
    out_ref[...] = _dot(h, w2_ref[...]) + b2_ref[...]


def _user_mlp(pu, W_u1, b_u1, W_u2, b_u2, interpret=False):
    return pl.pallas_call(
        _user_mlp_body,
        out_shape=jax.ShapeDtypeStruct((1, HID), jnp.float32),
        interpret=interpret,
    )(pu, W_u1, b_u1, W_u2, b_u2)


def _item_pass_body(item_ref, cent_ref, xx_ref, bias_ref, w1_ref, b1_ref,
                    w2_ref, b2_ref, uvc_ref, sim_ref, lab_ref):
    x = item_ref[...]
    s = _dot_rt(x, cent_ref[...])
    d = (xx_ref[...] - 2.0 * s) + bias_ref[...]
    dmin = jnp.min(d, axis=1, keepdims=True)
    ki = lax.broadcasted_iota(jnp.int32, (_B, K), 1)
    lab_ref[...] = jnp.min(jnp.where(d <= dmin, ki, K), axis=1, keepdims=True)

    h1 = jnp.maximum(_dot(x, w1_ref[...]) + b1_ref[...], 0.0)
    h = _dot(h1, w2_ref[...]) + b2_ref[...]
    nsq = jnp.sum(h * h, axis=1, keepdims=True)
    uvc = uvc_ref[...]
    num = _dot(h, uvc)
    unorm = jnp.sqrt(jnp.sum(uvc * uvc))
    sim_ref[...] = num / (jnp.sqrt(nsq) * unorm + 1e-8)


def _item_pass(item, cent, xx, bias, W_i1, b_i1, W_i2, b_i2, uvc,
               interpret=False):
    return pl.pallas_call(
        _item_pass_body,
        grid=(_NB,),
        in_specs=[
            pl.BlockSpec((_B, DIM), lambda j: (j, 0)),
            pl.BlockSpec((K, DIM), lambda j: (0, 0)),
            pl.BlockSpec((_B, 1), lambda j: (j, 0)),
            pl.BlockSpec((1, K), lambda j: (0, 0)),
            pl.BlockSpec((DIM, HID), lambda j: (0, 0)),
            pl.BlockSpec((1, HID), lambda j: (0, 0)),
            pl.BlockSpec((HID, HID), lambda j: (0, 0)),
            pl.BlockSpec((1, HID), lambda j: (0, 0)),
            pl.BlockSpec((HID, 1), lambda j: (0, 0)),
        ],
        out_specs=[
            pl.BlockSpec((_B, 1), lambda j: (j, 0)),
            pl.BlockSpec((_B, 1), lambda j: (j, 0)),
        ],
        out_shape=[
            jax.ShapeDtypeStruct((N_ITEMS, 1), jnp.float32),
            jax.ShapeDtypeStruct((N_ITEMS, 1), jnp.int32),
        ],
        compiler_params=pltpu.CompilerParams(
            dimension_semantics=("arbitrary",)),
        interpret=interpret,
    )(item, cent, xx, bias, W_i1, b_i1, W_i2, b_i2, uvc)


def _topk_body(sim_ref, lab_ref, out_ref, masked_s):
    fio = (lax.broadcasted_iota(jnp.int32, (_PROWS, 128), 0) * 128
           + lax.broadcasted_iota(jnp.int32, (_PROWS, 128), 1))
    sim = sim_ref[...]
    lab = lab_ref[...]
    for k in range(K):
        masked_s[...] = jnp.where(lab == k, sim, -1e30)

        def r_body(r, _, k=k):
            m = masked_s[...]
            mx = jnp.max(m)
            idx = jnp.min(jnp.where(m == mx, fio, jnp.int32(2 ** 30)))
            out_ref[k, r] = idx
            masked_s[...] = jnp.where(fio == idx, -jnp.inf, m)
            return 0

        lax.fori_loop(0, REC, r_body, 0)


def _topk(sim_p, lab_p, interpret=False):
    return pl.pallas_call(
        _topk_body,
        out_specs=pl.BlockSpec(memory_space=pltpu.SMEM),
        out_shape=jax.ShapeDtypeStruct((K, REC), jnp.int32),
        scratch_shapes=[pltpu.VMEM((_PROWS, 128), jnp.float32)],
        interpret=interpret,
    )(sim_p, lab_p)


def _sc_gather(item, idx2):
    mesh = plsc.VectorSubcoreMesh(core_axis_name="c", subcore_axis_name="s")

    @functools.partial(
        pl.kernel, mesh=mesh,
        out_type=jax.ShapeDtypeStruct((K * REC, DIM), jnp.float32),
        scratch_types=[
            pltpu.VMEM((2, 80), jnp.int32),
            pltpu.VMEM((K * REC, DIM), jnp.float32),
            pltpu.SemaphoreType.DMA,
        ],
    )
    def k(item_hbm, idx_hbm, out_hbm, idx_v, rows_v, sem):
        @pl.when(jnp.logical_and(lax.axis_index("c") == 0,
                                 lax.axis_index("s") == 0))
        def _():
            pltpu.sync_copy(idx_hbm, idx_v)
            for c in range(2):
                pltpu.async_copy(item_hbm.at[idx_v.at[c]],
                                 rows_v.at[pl.ds(c * 80, 80)], sem).wait()
            pltpu.sync_copy(rows_v, out_hbm)

    return k(item, idx2)


def _final_body(g_ref, w1_ref, b1_ref, w2_ref, b2_ref, uv_ref,
                wt1_ref, bt1_ref, wt2_ref, bt2_ref, out_ref, inter_s):
    g = g_ref[...]
    h1 = jnp.maximum(_dot(g, w1_ref[...]) + b1_ref[...], 0.0)
    h = _dot(h1, w2_ref[...]) + b2_ref[...]
    inter_s[:, :HID] = jnp.broadcast_to(uv_ref[...], (REC, HID))
    for k in range(K):
        inter_s[:, HID * (k + 1):HID * (k + 2)] = h[k * REC:(k + 1) * REC, :]
    inter = inter_s[...]
    t1 = jnp.maximum(_dot(inter, wt1_ref[...]) + bt1_ref[...], 0.0)
    out_ref[...] = _dot(t1, wt2_ref[...]) + bt2_ref[...]


def _final(g, W_i1, b_i1, W_i2, b_i2, uv, W_t1, b_t1, W_t2, b_t2,
           interpret=False):
    return pl.pallas_call(
        _final_body,
        out_shape=jax.ShapeDtypeStruct((REC, K), jnp.float32),
        scratch_shapes=[pltpu.VMEM((REC, HID * (K + 1)), jnp.float32)],
        interpret=interpret,
    )(g, W_i1, b_i1, W_i2, b_i2, uv, W_t1, b_t1, W_t2, b_t2)


def kernel(user, item, prompt_emb, W_u1, b_u1, W_u2, b_u2,
           W_i1, b_i1, W_i2, b_i2, W_t1, b_t1, W_t2, b_t2):
    f32 = jnp.float32
    cent = _kmeans_buckets(item)
    prompted = _prompted_user(user, cent, prompt_emb)
    pu = prompted.reshape(1, SEQ * 2 * DIM)
    uv = _user_mlp(pu, W_u1, b_u1.reshape(1, HID), W_u2, b_u2.reshape(1, HID))
    xx = jnp.sum(item * item, 1, keepdims=True)
    bias = jnp.sum(cent * cent, 1)[None, :]
    sim, lab = _item_pass(item, cent, xx, bias, W_i1, b_i1.reshape(1, HID),
                          W_i2, b_i2.reshape(1, HID), uv.reshape(HID, 1))
    pad = _NPAD - N_ITEMS
    sim_p = jnp.concatenate(
        [sim.reshape(-1), jnp.full((pad,), -3e30, f32)]).reshape(_PROWS, 128)
    lab_p = jnp.concatenate(
        [lab.reshape(-1), jnp.full((pad,), -1, jnp.int32)]).reshape(_PROWS, 128)
    idx = _topk(sim_p, lab_p)
    g = _sc_gather(item, idx.reshape(2, 80))
    res = _final(g, W_i1, b_i1.reshape(1, HID), W_i2, b_i2.reshape(1, HID),
                 uv, W_t1, b_t1.reshape(1, HID), W_t2,
                 b_t2.reshape(1, K))
    return res


# speedup vs baseline: 1.0078x; 1.0078x over previous
import functools

import jax
import jax.numpy as jnp
from jax import lax
from jax.experimental import pallas as pl
from jax.experimental.pallas import tpu as pltpu
from jax.experimental.pallas import tpu_sc as plsc

K = 8
HID = 512
DIM = 128
SEQ = 50
REC = 20
N_ITEMS = 100000
KM_ITERS = 5

_B = 2000
_NB = N_ITEMS // _B
_NPAD = 102400
_PROWS = _NPAD // 128

_DEF = lax.Precision.DEFAULT
_HI = lax.Precision.HIGHEST


def _dot(a, b, prec=_DEF):
    return lax.dot_general(a, b, (((1,), (0,)), ((), ())), precision=prec,
                           preferred_element_type=jnp.float32)


def _dot_rt(a, b, prec=_DEF):
    return lax.dot_general(a, b, (((1,), (1,)), ((), ())), precision=prec,
                           preferred_element_type=jnp.float32)


def _kmeans_buckets(items):
    cent = items[:K]
    for _ in range(KM_ITERS):
        d2 = (jnp.sum(items * items, 1, keepdims=True)
              - 2.0 * items @ cent.T
              + jnp.sum(cent * cent, 1)[None, :])
        lab = jnp.argmin(d2, axis=1)
        sums = jax.ops.segment_sum(items, lab, num_segments=K)
        oh = (lab[:, None] == jnp.arange(K)[None, :]).astype(jnp.float32)
        cnt = jnp.sum(oh, axis=0)
        cent = sums / jnp.maximum(cnt, 1.0)[:, None]
    d2 = (jnp.sum(items * items, 1, keepdims=True)
          - 2.0 * items @ cent.T
          + jnp.sum(cent * cent, 1)[None, :])
    lab = jnp.argmin(d2, axis=1)
    return lab, cent


def _prompt_body(user_ref, cent_ref, pe_ref, out_ref):
    u = user_ref[...]
    c = cent_ref[...]
    s = _dot_rt(u, c)
    bias = lax.dot_general(jnp.ones((1, DIM), jnp.float32), c * c,
                           (((1,), (1,)), ((), ())), precision=_HI,
                           preferred_element_type=jnp.float32)
    xx = jnp.sum(u * u, axis=1, keepdims=True)
    d = (xx - 2.0 * s) + bias
    dmin = jnp.min(d, axis=1, keepdims=True)
    ki = lax.broadcasted_iota(jnp.int32, (SEQ, K), 1)
    idx = jnp.min(jnp.where(d <= dmin, ki, K), axis=1, keepdims=True)
    pr = jnp.zeros((SEQ, DIM), jnp.float32)
    for k in range(K):
        pr = jnp.where(idx == k, pe_ref[k:k + 1, :], pr)
    out_ref[:, :DIM] = u
    out_ref[:, DIM:] = pr


def _prompted_user(user, cent, prompt_emb, interpret=False):
    return pl.pallas_call(
        _prompt_body,
        out_shape=jax.ShapeDtypeStruct((SEQ, 2 * DIM), jnp.float32),
        interpret=interpret,
    )(user, cent, prompt_emb)


def _user_mlp_body(pu_ref, w1_ref, b1_ref, w2_ref, b2_ref, out_ref):
    h = jnp.maximum(_dot(pu_ref[...], w1_ref[...]) + b1_ref[...], 0.0)
    out_ref[...] = _dot(h, w2_ref[...]) + b2_ref[...]


def _user_mlp(pu, W_u1, b_u1, W_u2, b_u2, interpret=False):
    return pl.pallas_call(
        _user_mlp_body,
        out_shape=jax.ShapeDtypeStruct((1, HID), jnp.float32),
        interpret=interpret,
    )(pu, W_u1, b_u1, W_u2, b_u2)


def _item_pass_body(item_ref, w1_ref, b1_ref, w2_ref, b2_ref, uvc_ref,
                    sim_ref):
    x = item_ref[...]
    h1 = jnp.maximum(_dot(x, w1_ref[...]) + b1_ref[...], 0.0)
    h = _dot(h1, w2_ref[...]) + b2_ref[...]
    nsq = jnp.sum(h * h, axis=1, keepdims=True)
    uvc = uvc_ref[...]
    num = _dot(h, uvc)
    unorm = jnp.sqrt(jnp.sum(uvc * uvc))
    sim_ref[...] = num / (jnp.sqrt(nsq) * unorm + 1e-8)


def _item_pass(item, W_i1, b_i1, W_i2, b_i2, uvc, interpret=False):
    return pl.pallas_call(
        _item_pass_body,
        grid=(_NB,),
        in_specs=[
            pl.BlockSpec((_B, DIM), lambda j: (j, 0)),
            pl.BlockSpec((DIM, HID), lambda j: (0, 0)),
            pl.BlockSpec((1, HID), lambda j: (0, 0)),
            pl.BlockSpec((HID, HID), lambda j: (0, 0)),
            pl.BlockSpec((1, HID), lambda j: (0, 0)),
            pl.BlockSpec((HID, 1), lambda j: (0, 0)),
        ],
        out_specs=pl.BlockSpec((_B, 1), lambda j: (j, 0)),
        out_shape=jax.ShapeDtypeStruct((N_ITEMS, 1), jnp.float32),
        compiler_params=pltpu.CompilerParams(
            dimension_semantics=("arbitrary",)),
        interpret=interpret,
    )(item, W_i1, b_i1, W_i2, b_i2, uvc)


def _topk_body(sim_ref, lab_ref, out_ref, masked_s):
    fio = (lax.broadcasted_iota(jnp.int32, (_PROWS, 128), 0) * 128
           + lax.broadcasted_iota(jnp.int32, (_PROWS, 128), 1))
    sim = sim_ref[...]
    lab = lab_ref[...]
    for k in range(K):
        masked_s[...] = jnp.where(lab == k, sim, -1e30)

        def r_body(r, _, k=k):
            m = masked_s[...]
            mx = jnp.max(m)
            idx = jnp.min(jnp.where(m == mx, fio, jnp.int32(2 ** 30)))
            out_ref[k, r] = idx
            masked_s[...] = jnp.where(fio == idx, -jnp.inf, m)
            return 0

        lax.fori_loop(0, REC, r_body, 0)


def _topk(sim_p, lab_p, interpret=False):
    return pl.pallas_call(
        _topk_body,
        out_specs=pl.BlockSpec(memory_space=pltpu.SMEM),
        out_shape=jax.ShapeDtypeStruct((K, REC), jnp.int32),
        scratch_shapes=[pltpu.VMEM((_PROWS, 128), jnp.float32)],
        interpret=interpret,
    )(sim_p, lab_p)


def _sc_gather(item, idx2):
    mesh = plsc.VectorSubcoreMesh(core_axis_name="c", subcore_axis_name="s")

    @functools.partial(
        pl.kernel, mesh=mesh,
        out_type=jax.ShapeDtypeStruct((K * REC, DIM), jnp.float32),
        scratch_types=[
            pltpu.VMEM((2, 80), jnp.int32),
            pltpu.VMEM((K * REC, DIM), jnp.float32),
            pltpu.SemaphoreType.DMA,
        ],
    )
    def k(item_hbm, idx_hbm, out_hbm, idx_v, rows_v, sem):
        @pl.when(jnp.logical_and(lax.axis_index("c") == 0,
                                 lax.axis_index("s") == 0))
        def _():
            pltpu.sync_copy(idx_hbm, idx_v)
            for c in range(2):
                pltpu.async_copy(item_hbm.at[idx_v.at[c]],
                                 rows_v.at[pl.ds(c * 80, 80)], sem).wait()
            pltpu.sync_copy(rows_v, out_hbm)

    return k(item, idx2)


def _final_body(g_ref, w1_ref, b1_ref, w2_ref, b2_ref, uv_ref,
                wt1_ref, bt1_ref, wt2_ref, bt2_ref, out_ref, inter_s):
    g = g_ref[...]
    h1 = jnp.maximum(_dot(g, w1_ref[...]) + b1_ref[...], 0.0)
    h = _dot(h1, w2_ref[...]) + b2_ref[...]
    inter_s[:, :HID] = jnp.broadcast_to(uv_ref[...], (REC, HID))
    for k in range(K):
        inter_s[:, HID * (k + 1):HID * (k + 2)] = h[k * REC:(k + 1) * REC, :]
    inter = inter_s[...]
    t1 = jnp.maximum(_dot(inter, wt1_ref[...]) + bt1_ref[...], 0.0)
    out_ref[...] = _dot(t1, wt2_ref[...]) + bt2_ref[...]


def _final(g, W_i1, b_i1, W_i2, b_i2, uv, W_t1, b_t1, W_t2, b_t2,
           interpret=False):
    return pl.pallas_call(
        _final_body,
        out_shape=jax.ShapeDtypeStruct((REC, K), jnp.float32),
        scratch_shapes=[pltpu.VMEM((REC, HID * (K + 1)), jnp.float32)],
        interpret=interpret,
    )(g, W_i1, b_i1, W_i2, b_i2, uv, W_t1, b_t1, W_t2, b_t2)


def kernel(user, item, prompt_emb, W_u1, b_u1, W_u2, b_u2,
           W_i1, b_i1, W_i2, b_i2, W_t1, b_t1, W_t2, b_t2):
    f32 = jnp.float32
    lab, cent = _kmeans_buckets(item)
    prompted = _prompted_user(user, cent, prompt_emb)
    pu = prompted.reshape(1, SEQ * 2 * DIM)
    uv = _user_mlp(pu, W_u1, b_u1.reshape(1, HID), W_u2, b_u2.reshape(1, HID))
    sim = _item_pass(item, W_i1, b_i1.reshape(1, HID),
                     W_i2, b_i2.reshape(1, HID), uv.reshape(HID, 1))
    pad = _NPAD - N_ITEMS
    sim_p = jnp.concatenate(
        [sim.reshape(-1), jnp.full((pad,), -3e30, f32)]).reshape(_PROWS, 128)
    lab_p = jnp.concatenate(
        [lab.astype(jnp.int32), jnp.full((pad,), -1, jnp.int32)]).reshape(_PROWS, 128)
    idx = _topk(sim_p, lab_p)
    g = _sc_gather(item, idx.reshape(2, 80))
    res = _final(g, W_i1, b_i1.reshape(1, HID), W_i2, b_i2.reshape(1, HID),
                 uv, W_t1, b_t1.reshape(1, HID), W_t2,
                 b_t2.reshape(1, K))
    return res
